# baseline (speedup 1.0000x reference)
import functools

import jax
import jax.numpy as jnp
from jax import lax
from jax.experimental import pallas as pl
from jax.experimental.pallas import tpu as pltpu

M_BLK = 256
F_BLK = 256
_E = 8
_K = 2

_C_ACT, _C_WAIT, _C_SLOT, _C_ISS, _C_ISLOT, _C_IE, _C_IF, _C_BE = range(8)

_DT = 4


def _w2t_kernel(w2_ref, w2s_ref, out_ref):
    wt = w2_ref[0] * w2s_ref[0]
    out_ref[0, 0] = wt.T.astype(jnp.bfloat16)


def _moe_mlp_kernel(sched_ref, xs_ref, w1, w1s, w3, w3s, w2t, out_ref,
                    r1, r1s, r3, r3s, r2, w1d, w3d, sems):
    f = pl.program_id(0)
    m = pl.program_id(1)
    nb = pl.num_programs(1)
    s = f * nb + m

    def copies(e_, f_, sl):
        fo = f_ * F_BLK
        return [
            pltpu.make_async_copy(w1.at[e_, pl.ds(fo, F_BLK), :],
                                  r1.at[sl], sems.at[sl]),
            pltpu.make_async_copy(w1s.at[e_, pl.ds(fo, F_BLK), :],
                                  r1s.at[sl], sems.at[sl]),
            pltpu.make_async_copy(w3.at[e_, pl.ds(fo, F_BLK), :],
                                  r3.at[sl], sems.at[sl]),
            pltpu.make_async_copy(w3s.at[e_, pl.ds(fo, F_BLK), :],
                                  r3s.at[sl], sems.at[sl]),
            pltpu.make_async_copy(w2t.at[e_, :, pl.ds(fo, F_BLK), :],
                                  r2.at[sl], sems.at[sl]),
        ]

    @pl.when(s == 0)
    def _():
        for c in copies(sched_ref[0, _C_BE], 0, 0):
            c.start()

    @pl.when(sched_ref[s, _C_ACT] == 1)
    def _():
        @pl.when(sched_ref[s, _C_ISS] == 1)
        def _():
            for c in copies(sched_ref[s, _C_IE], sched_ref[s, _C_IF],
                            sched_ref[s, _C_ISLOT]):
                c.start()

        @pl.when(sched_ref[s, _C_WAIT] == 1)
        def _():
            sl = sched_ref[s, _C_SLOT]
            for c in copies(0, 0, sl):
                c.wait()
            w1d[...] = (r1[sl] * r1s[sl]).astype(jnp.bfloat16)
            w3d[...] = (r3[sl] * r3s[sl]).astype(jnp.bfloat16)

        sl = sched_ref[s, _C_SLOT]
        x = xs_ref[pl.ds(m * M_BLK, M_BLK), :]
        dn = (((1,), (1,)), ((), ()))
        dnn = (((1,), (0,)), ((), ()))
        h1 = lax.dot_general(x, w1d[...], dn,
                             preferred_element_type=jnp.float32)
        h3 = lax.dot_general(x, w3d[...], dn,
                             preferred_element_type=jnp.float32)
        h = h1 * jax.nn.sigmoid(h1) * h3
        hb = h.astype(jnp.bfloat16)
        sl_m = pl.ds(m * M_BLK, M_BLK)
        dtile = xs_ref.shape[1] // _DT
        for dt in range(_DT):
            c = lax.dot_general(hb, r2[sl, dt], dnn,
                                preferred_element_type=jnp.float32)

            @pl.when(f == 0)
            def _(c=c, dt=dt):
                out_ref[sl_m, dt * dtile:(dt + 1) * dtile] = c

            @pl.when(f != 0)
            def _(c=c, dt=dt):
                out_ref[sl_m, dt * dtile:(dt + 1) * dtile] += c


@functools.partial(jax.jit, static_argnames=())
def kernel(hidden_states, gate_w, w1, w1_scale, w3, w3_scale, w2, w2_scale):
    b, s_len, d = hidden_states.shape
    e, f_dim, _ = w1.shape
    x = hidden_states.reshape(-1, d)
    t = x.shape[0]
    a = t * _K
    nb = (a + _E * (M_BLK - 1)) // M_BLK + 1
    p = nb * M_BLK
    nf = f_dim // F_BLK
    n_steps = nf * nb

    router_logits = x @ gate_w.T
    scores = jax.nn.sigmoid(router_logits)
    top_vals, top_idx = lax.top_k(scores, _K)
    routing_w = top_vals / jnp.sum(top_vals, axis=-1, keepdims=True)

    e_flat = top_idx.reshape(-1).astype(jnp.int32)
    oh = (e_flat[:, None] == jnp.arange(_E, dtype=jnp.int32)[None, :]
          ).astype(jnp.int32)
    csum = jnp.cumsum(oh, axis=0)
    counts = csum[-1]
    rank = jnp.take_along_axis(csum - oh, e_flat[:, None], axis=1)[:, 0]
    padded = ((counts + M_BLK - 1) // M_BLK) * M_BLK
    pad_cum = jnp.cumsum(padded)
    starts = pad_cum - padded
    dst = starts[e_flat] + rank
    tok_of_a = jnp.arange(a, dtype=jnp.int32) // _K
    src = jnp.zeros((p,), jnp.int32).at[dst].set(tok_of_a)
    block_expert = jnp.minimum(
        jnp.searchsorted(pad_cum, jnp.arange(nb, dtype=jnp.int32) * M_BLK,
                         side="right").astype(jnp.int32), _E - 1)
    n_active = (pad_cum[-1] // M_BLK).astype(jnp.int32)

    steps = jnp.arange(n_steps, dtype=jnp.int32)
    m_s = steps % nb
    f_s = steps // nb
    be_s = block_expert[m_s]
    act_s = (m_s < n_active).astype(jnp.int32)
    run_start = (act_s == 1) & ((m_s == 0) | (be_s != block_expert[m_s - 1]))
    r_s = jnp.cumsum(run_start.astype(jnp.int32)) - 1
    nruns = jnp.sum(run_start.astype(jnp.int32))
    maxr = nf * _E
    run_e = jnp.zeros((maxr + 1,), jnp.int32).at[
        jnp.where(run_start, r_s, maxr)].set(be_s)
    run_f = jnp.zeros((maxr + 1,), jnp.int32).at[
        jnp.where(run_start, r_s, maxr)].set(f_s)
    has_next = (r_s + 1) < nruns
    iss_s = run_start & has_next
    sched = jnp.stack([
        act_s,
        run_start.astype(jnp.int32),
        r_s % 2,
        iss_s.astype(jnp.int32),
        (r_s + 1) % 2,
        run_e[jnp.minimum(r_s + 1, maxr)],
        run_f[jnp.minimum(r_s + 1, maxr)],
        be_s,
    ], axis=1)

    xs = x[src].astype(jnp.bfloat16)
    dtile = d // _DT
    w2t = pl.pallas_call(
        _w2t_kernel,
        grid=(e, _DT),
        in_specs=[
            pl.BlockSpec((1, dtile, f_dim), lambda ei, ti: (ei, ti, 0)),
            pl.BlockSpec((1, dtile, f_dim), lambda ei, ti: (ei, ti, 0)),
        ],
        out_specs=pl.BlockSpec((1, 1, f_dim, dtile),
                               lambda ei, ti: (ei, ti, 0, 0)),
        out_shape=jax.ShapeDtypeStruct((e, _DT, f_dim, dtile), jnp.bfloat16),
    )(w2, w2_scale)

    rows = pl.pallas_call(
        _moe_mlp_kernel,
        grid=(nf, nb),
        in_specs=[
            pl.BlockSpec(memory_space=pltpu.SMEM),
            pl.BlockSpec((p, d), lambda f, m: (0, 0)),
            pl.BlockSpec(memory_space=pltpu.HBM),
            pl.BlockSpec(memory_space=pltpu.HBM),
            pl.BlockSpec(memory_space=pltpu.HBM),
            pl.BlockSpec(memory_space=pltpu.HBM),
            pl.BlockSpec(memory_space=pltpu.HBM),
        ],
        out_specs=pl.BlockSpec((p, d), lambda f, m: (0, 0)),
        scratch_shapes=[
            pltpu.VMEM((2, F_BLK, d), jnp.float32),
            pltpu.VMEM((2, F_BLK, d), jnp.float32),
            pltpu.VMEM((2, F_BLK, d), jnp.float32),
            pltpu.VMEM((2, F_BLK, d), jnp.float32),
            pltpu.VMEM((2, _DT, F_BLK, d // _DT), jnp.bfloat16),
            pltpu.VMEM((F_BLK, d), jnp.bfloat16),
            pltpu.VMEM((F_BLK, d), jnp.bfloat16),
            pltpu.SemaphoreType.DMA((2,)),
        ],
        out_shape=jax.ShapeDtypeStruct((p, d), jnp.float32),
    )(sched, xs, w1, w1_scale, w3, w3_scale, w2t)

    d0 = dst[0::2]
    d1 = dst[1::2]
    y = rows[d0] * routing_w[:, :1] + rows[d1] * routing_w[:, 1:]
    return y.reshape(b, s_len, d)


# speedup vs baseline: 1.5087x; 1.5087x over previous
import functools

import jax
import jax.numpy as jnp
from jax import lax
from jax.experimental import pallas as pl
from jax.experimental.pallas import tpu as pltpu

M_BLK = 256
F_BLK = 256
_E = 8
_K = 2


def _moe_mlp_kernel(meta_ref, xs_ref, w1, w1s, w3, w3s, w2, w2s, out_ref,
                    r1, r1s, r3, r3s, r2, r2s, w1d, w3d, w2d, state, sems):
    f = pl.program_id(0)
    m = pl.program_id(1)
    nb = pl.num_programs(1)
    nf = pl.num_programs(0)
    n_active = meta_ref[nb]

    def copies(e_, f_, sl):
        fo = f_ * F_BLK
        return [
            pltpu.make_async_copy(w1.at[e_, pl.ds(fo, F_BLK), :],
                                  r1.at[sl], sems.at[sl]),
            pltpu.make_async_copy(w1s.at[e_, pl.ds(fo, F_BLK), :],
                                  r1s.at[sl], sems.at[sl]),
            pltpu.make_async_copy(w3.at[e_, pl.ds(fo, F_BLK), :],
                                  r3.at[sl], sems.at[sl]),
            pltpu.make_async_copy(w3s.at[e_, pl.ds(fo, F_BLK), :],
                                  r3s.at[sl], sems.at[sl]),
            pltpu.make_async_copy(w2.at[e_, :, pl.ds(fo, F_BLK)],
                                  r2.at[sl], sems.at[sl]),
            pltpu.make_async_copy(w2s.at[e_, :, pl.ds(fo, F_BLK)],
                                  r2s.at[sl], sems.at[sl]),
        ]

    @pl.when(jnp.logical_and(f == 0, m == 0))
    def _():
        state[0] = 0
        for c in copies(meta_ref[0], 0, 0):
            c.start()

    @pl.when(m < n_active)
    def _():
        be_m = meta_ref[m]
        prev = meta_ref[jnp.maximum(m - 1, 0)]
        is_start = jnp.logical_or(m == 0, be_m != prev)

        @pl.when(is_start)
        def _():
            r = state[0]
            slot = lax.rem(r, 2)
            nxt = lax.while_loop(
                lambda i: jnp.logical_and(i < n_active, meta_ref[i] == be_m),
                lambda i: i + 1, m + 1)
            in_sweep = nxt < n_active
            ne = jnp.where(in_sweep, meta_ref[jnp.minimum(nxt, nb - 1)],
                           meta_ref[0])
            nf_ = jnp.where(in_sweep, f, f + 1)

            @pl.when(jnp.logical_or(in_sweep, f + 1 < nf))
            def _():
                for c in copies(ne, nf_, lax.rem(r + 1, 2)):
                    c.start()

            for c in copies(0, 0, slot):
                c.wait()
            w1d[...] = (r1[slot] * r1s[slot]).astype(jnp.bfloat16)
            w3d[...] = (r3[slot] * r3s[slot]).astype(jnp.bfloat16)
            w2d[...] = (r2[slot] * r2s[slot]).astype(jnp.bfloat16)
            state[0] = r + 1

        x = xs_ref[pl.ds(m * M_BLK, M_BLK), :]
        dn = (((1,), (1,)), ((), ()))
        h1 = lax.dot_general(x, w1d[...], dn,
                             preferred_element_type=jnp.float32)
        h3 = lax.dot_general(x, w3d[...], dn,
                             preferred_element_type=jnp.float32)
        h = h1 * jax.nn.sigmoid(h1) * h3
        contrib = lax.dot_general(h.astype(jnp.bfloat16), w2d[...], dn,
                                  preferred_element_type=jnp.float32)
        sl_m = pl.ds(m * M_BLK, M_BLK)

        @pl.when(f == 0)
        def _():
            out_ref[sl_m, :] = contrib

        @pl.when(f != 0)
        def _():
            out_ref[sl_m, :] += contrib


@functools.partial(jax.jit, static_argnames=())
def kernel(hidden_states, gate_w, w1, w1_scale, w3, w3_scale, w2, w2_scale):
    b, s_len, d = hidden_states.shape
    e, f_dim, _ = w1.shape
    x = hidden_states.reshape(-1, d)
    t = x.shape[0]
    a = t * _K
    nb = (a + _E * (M_BLK - 1)) // M_BLK + 1
    p = nb * M_BLK
    nf = f_dim // F_BLK

    router_logits = x @ gate_w.T
    scores = jax.nn.sigmoid(router_logits)
    top_vals, top_idx = lax.top_k(scores, _K)
    routing_w = top_vals / jnp.sum(top_vals, axis=-1, keepdims=True)

    e_flat = top_idx.reshape(-1).astype(jnp.int32)
    oh = (e_flat[:, None] == jnp.arange(_E, dtype=jnp.int32)[None, :]
          ).astype(jnp.int32)
    csum = jnp.cumsum(oh, axis=0)
    counts = csum[-1]
    rank = jnp.take_along_axis(csum - oh, e_flat[:, None], axis=1)[:, 0]
    padded = ((counts + M_BLK - 1) // M_BLK) * M_BLK
    pad_cum = jnp.cumsum(padded)
    starts = pad_cum - padded
    dst = starts[e_flat] + rank
    tok_of_a = jnp.arange(a, dtype=jnp.int32) // _K
    src = jnp.zeros((p,), jnp.int32).at[dst].set(tok_of_a)
    block_expert = jnp.minimum(
        jnp.searchsorted(pad_cum, jnp.arange(nb, dtype=jnp.int32) * M_BLK,
                         side="right").astype(jnp.int32), _E - 1)
    n_active = (pad_cum[-1] // M_BLK).astype(jnp.int32)
    meta = jnp.concatenate([block_expert, n_active[None]])

    xs = x[src].astype(jnp.bfloat16)

    rows = pl.pallas_call(
        _moe_mlp_kernel,
        grid=(nf, nb),
        in_specs=[
            pl.BlockSpec(memory_space=pltpu.SMEM),
            pl.BlockSpec((p, d), lambda f, m: (0, 0)),
            pl.BlockSpec(memory_space=pltpu.HBM),
            pl.BlockSpec(memory_space=pltpu.HBM),
            pl.BlockSpec(memory_space=pltpu.HBM),
            pl.BlockSpec(memory_space=pltpu.HBM),
            pl.BlockSpec(memory_space=pltpu.HBM),
            pl.BlockSpec(memory_space=pltpu.HBM),
        ],
        out_specs=pl.BlockSpec((p, d), lambda f, m: (0, 0)),
        scratch_shapes=[
            pltpu.VMEM((2, F_BLK, d), jnp.float32),
            pltpu.VMEM((2, F_BLK, d), jnp.float32),
            pltpu.VMEM((2, F_BLK, d), jnp.float32),
            pltpu.VMEM((2, F_BLK, d), jnp.float32),
            pltpu.VMEM((2, d, F_BLK), jnp.float32),
            pltpu.VMEM((2, d, F_BLK), jnp.float32),
            pltpu.VMEM((F_BLK, d), jnp.bfloat16),
            pltpu.VMEM((F_BLK, d), jnp.bfloat16),
            pltpu.VMEM((d, F_BLK), jnp.bfloat16),
            pltpu.SMEM((2,), jnp.int32),
            pltpu.SemaphoreType.DMA((2,)),
        ],
        out_shape=jax.ShapeDtypeStruct((p, d), jnp.float32),
    )(meta, xs, w1, w1_scale, w3, w3_scale, w2, w2_scale)

    d0 = dst[0::2]
    d1 = dst[1::2]
    y = rows[d0] * routing_w[:, :1] + rows[d1] * routing_w[:, 1:]
    return y.reshape(b, s_len, d)
